# baseline (speedup 1.0000x reference)
import functools

import jax
import jax.numpy as jnp
import numpy as np
from jax import lax
from jax.experimental import pallas as pl
from jax.experimental.pallas import tpu as pltpu
from jax.experimental.pallas import tpu_sc as plsc

D = 8
DP = 16
DD = D * D
NC = 2
NS = 16
NW = NC * NS
CHUNK = 1000

_GDN = lax.GatherDimensionNumbers(
    offset_dims=(), collapsed_slice_dims=(0,), start_index_map=(0,)
)


def _lane_shuffle(v, idx):
    return lax.gather(
        v, idx.reshape(16, 1), _GDN, slice_sizes=(1,),
        mode=lax.GatherScatterMode.PROMISE_IN_BOUNDS,
    )


def _make(n_edges):
    e_per_w = n_edges // NW
    n_chunks = e_per_w // CHUNK
    mesh = plsc.VectorSubcoreMesh(core_axis_name="c", subcore_axis_name="s")

    def body(si_hbm, ri_hbm, s_hbm, r_hbm, out_hbm,
             sidx_v, ridx_v, srow_v, rrow_v, out_v, sem_s, sem_r):
        wid = lax.axis_index("s") * NC + lax.axis_index("c")
        base_w = wid * e_per_w
        lane = lax.iota(jnp.int32, 16)
        r_rep_idx = lane % D
        s_pat_idx = [lane // D + 2 * k for k in range(4)]

        def chunk_body(k, carry):
            base = base_w + k * CHUNK
            pltpu.sync_copy(si_hbm.at[pl.ds(base, CHUNK)], sidx_v)
            pltpu.sync_copy(ri_hbm.at[pl.ds(base, CHUNK)], ridx_v)
            cs = pltpu.async_copy(s_hbm.at[sidx_v], srow_v, sem_s)
            cr = pltpu.async_copy(r_hbm.at[ridx_v], rrow_v, sem_r)
            cs.wait()
            cr.wait()

            def edge_body(c, c2):
                s_vec = srow_v[c]
                r_vec = rrow_v[c]
                r_rep = _lane_shuffle(r_vec, r_rep_idx)
                c64 = c * DD
                for k in range(4):
                    s_pat = _lane_shuffle(s_vec, s_pat_idx[k])
                    out_v[pl.ds(c64 + k * 16, 16)] = s_pat * r_rep
                return c2

            lax.fori_loop(0, CHUNK, edge_body, 0, unroll=4)
            pltpu.sync_copy(out_v, out_hbm.at[pl.ds(base * DD, CHUNK * DD)])
            return carry

        lax.fori_loop(0, n_chunks, chunk_body, 0, unroll=False)

    return pl.kernel(
        body,
        out_type=jax.ShapeDtypeStruct((n_edges * DD,), jnp.float32),
        mesh=mesh,
        compiler_params=pltpu.CompilerParams(
            needs_layout_passes=False, use_tc_tiling_on_sc=False
        ),
        scratch_types=[
            pltpu.VMEM((CHUNK,), jnp.int32),
            pltpu.VMEM((CHUNK,), jnp.int32),
            pltpu.VMEM((CHUNK, DP), jnp.float32),
            pltpu.VMEM((CHUNK, DP), jnp.float32),
            pltpu.VMEM((CHUNK * DD,), jnp.float32),
            pltpu.SemaphoreType.DMA,
            pltpu.SemaphoreType.DMA,
        ],
    )


def kernel(edge_index, node_type_s, node_type_r=None):
    if node_type_r is None:
        node_type_r = node_type_s
    n_edges = edge_index.shape[1]
    pad = ((0, 0), (0, DP - D))
    s_p = jnp.pad(node_type_s, pad)
    r_p = jnp.pad(node_type_r, pad)
    f = _make(n_edges)
    out_flat = f(edge_index[0], edge_index[1], s_p, r_p)
    return out_flat.reshape(n_edges, DD)


# speedup vs baseline: 1.1942x; 1.1942x over previous
import functools

import jax
import jax.numpy as jnp
import numpy as np
from jax import lax
from jax.experimental import pallas as pl
from jax.experimental.pallas import tpu as pltpu
from jax.experimental.pallas import tpu_sc as plsc

D = 8
DP = 16
DD = D * D
NC = 2
NS = 16
NW = NC * NS
CHUNK = 400

_GDN = lax.GatherDimensionNumbers(
    offset_dims=(), collapsed_slice_dims=(0,), start_index_map=(0,)
)


def _lane_shuffle(v, idx):
    return lax.gather(
        v, idx.reshape(16, 1), _GDN, slice_sizes=(1,),
        mode=lax.GatherScatterMode.PROMISE_IN_BOUNDS,
    )


def _make(n_edges):
    e_per_w = n_edges // NW
    n_chunks = e_per_w // CHUNK
    mesh = plsc.VectorSubcoreMesh(core_axis_name="c", subcore_axis_name="s")

    def body(si_hbm, ri_hbm, s_hbm, r_hbm, out_hbm,
             sidx_v, ridx_v, srow_v, rrow_v, out_v,
             sem_si, sem_ri, sem_sg, sem_rg, sem_out):
        wid = lax.axis_index("s") * NC + lax.axis_index("c")
        base_w = wid * e_per_w
        lane = lax.iota(jnp.int32, 16)
        r_rep_idx = lane % D
        s_pat_idx = [lane // D + 2 * k for k in range(4)]

        def idx_start(k, b):
            base = base_w + k * CHUNK
            pltpu.make_async_copy(
                si_hbm.at[pl.ds(base, CHUNK)], sidx_v.at[b], sem_si.at[b]
            ).start()
            pltpu.make_async_copy(
                ri_hbm.at[pl.ds(base, CHUNK)], ridx_v.at[b], sem_ri.at[b]
            ).start()

        def idx_wait(b):
            pltpu.make_async_copy(
                si_hbm.at[pl.ds(0, CHUNK)], sidx_v.at[b], sem_si.at[b]
            ).wait()
            pltpu.make_async_copy(
                ri_hbm.at[pl.ds(0, CHUNK)], ridx_v.at[b], sem_ri.at[b]
            ).wait()

        def gather_start(b):
            pltpu.make_async_copy(
                s_hbm.at[sidx_v.at[b]], srow_v.at[b], sem_sg.at[b]
            ).start()
            pltpu.make_async_copy(
                r_hbm.at[ridx_v.at[b]], rrow_v.at[b], sem_rg.at[b]
            ).start()

        def gather_wait(b):
            pltpu.make_async_copy(
                s_hbm.at[sidx_v.at[b]], srow_v.at[b], sem_sg.at[b]
            ).wait()
            pltpu.make_async_copy(
                r_hbm.at[ridx_v.at[b]], rrow_v.at[b], sem_rg.at[b]
            ).wait()

        def out_start(k, b):
            base = base_w + k * CHUNK
            pltpu.make_async_copy(
                out_v.at[b], out_hbm.at[pl.ds(base * DD, CHUNK * DD)], sem_out.at[b]
            ).start()

        def out_wait(b):
            pltpu.make_async_copy(
                out_v.at[b], out_hbm.at[pl.ds(0, CHUNK * DD)], sem_out.at[b]
            ).wait()

        def compute(b):
            def edge_body(c, c2):
                s_vec = srow_v[b, c]
                r_vec = rrow_v[b, c]
                r_rep = _lane_shuffle(r_vec, r_rep_idx)
                c64 = c * DD
                for k in range(4):
                    s_pat = _lane_shuffle(s_vec, s_pat_idx[k])
                    out_v[b, pl.ds(c64 + k * 16, 16)] = s_pat * r_rep
                return c2

            lax.fori_loop(0, CHUNK, edge_body, 0, unroll=8)

        idx_start(0, 0)
        idx_start(1, 1)
        idx_wait(0)
        gather_start(0)

        def pair_body(k2, carry):
            for b in (0, 1):
                k = 2 * k2 + b

                @pl.when(k < n_chunks)
                def _():
                    @pl.when(k + 1 < n_chunks)
                    def _():
                        idx_wait(1 - b)
                        gather_start(1 - b)

                    gather_wait(b)

                    @pl.when(k + 2 < n_chunks)
                    def _():
                        idx_start(k + 2, b)

                    @pl.when(k >= 2)
                    def _():
                        out_wait(b)

                    compute(b)
                    out_start(k, b)

            return carry

        lax.fori_loop(0, (n_chunks + 1) // 2, pair_body, 0, unroll=False)
        out_wait(0)
        out_wait(1)

    return pl.kernel(
        body,
        out_type=jax.ShapeDtypeStruct((n_edges * DD,), jnp.float32),
        mesh=mesh,
        compiler_params=pltpu.CompilerParams(
            needs_layout_passes=False, use_tc_tiling_on_sc=False
        ),
        scratch_types=[
            pltpu.VMEM((2, CHUNK), jnp.int32),
            pltpu.VMEM((2, CHUNK), jnp.int32),
            pltpu.VMEM((2, CHUNK, DP), jnp.float32),
            pltpu.VMEM((2, CHUNK, DP), jnp.float32),
            pltpu.VMEM((2, CHUNK * DD), jnp.float32),
            pltpu.SemaphoreType.DMA((2,)),
            pltpu.SemaphoreType.DMA((2,)),
            pltpu.SemaphoreType.DMA((2,)),
            pltpu.SemaphoreType.DMA((2,)),
            pltpu.SemaphoreType.DMA((2,)),
        ],
    )


def kernel(edge_index, node_type_s, node_type_r=None):
    if node_type_r is None:
        node_type_r = node_type_s
    n_edges = edge_index.shape[1]
    pad = ((0, 0), (0, DP - D))
    s_p = jnp.pad(node_type_s, pad)
    r_p = jnp.pad(node_type_r, pad)
    f = _make(n_edges)
    out_flat = f(edge_index[0], edge_index[1], s_p, r_p)
    return out_flat.reshape(n_edges, DD)


# speedup vs baseline: 1.3675x; 1.1451x over previous
import functools

import jax
import jax.numpy as jnp
import numpy as np
from jax import lax
from jax.experimental import pallas as pl
from jax.experimental.pallas import tpu as pltpu
from jax.experimental.pallas import tpu_sc as plsc

D = 8
DP = 16
DD = D * D
NC = 2
NS = 16
NW = NC * NS
CHUNK = 400

_GDN = lax.GatherDimensionNumbers(
    offset_dims=(), collapsed_slice_dims=(0,), start_index_map=(0,)
)


def _lane_shuffle(v, idx):
    return lax.gather(
        v, idx.reshape(16, 1), _GDN, slice_sizes=(1,),
        mode=lax.GatherScatterMode.PROMISE_IN_BOUNDS,
    )


def _make(n_edges):
    e_per_w = n_edges // NW
    n_chunks = e_per_w // CHUNK
    mesh = plsc.VectorSubcoreMesh(core_axis_name="c", subcore_axis_name="s")

    def body(si_hbm, ri_hbm, s_hbm, r_hbm, out_hbm,
             sidx_v, ridx_v, srow_v, rrow_v, out_v,
             sem_si, sem_ri, sem_sg, sem_rg, sem_out):
        wid = lax.axis_index("s") * NC + lax.axis_index("c")
        base_w = wid * e_per_w
        lane = lax.iota(jnp.int32, 16)
        s_pat_idx = [lane // D + 2 * k for k in range(4)]

        def idx_start(k, b):
            base = base_w + k * CHUNK
            pltpu.make_async_copy(
                si_hbm.at[pl.ds(base, CHUNK)], sidx_v.at[b], sem_si.at[b]
            ).start()
            pltpu.make_async_copy(
                ri_hbm.at[pl.ds(base, CHUNK)], ridx_v.at[b], sem_ri.at[b]
            ).start()

        def idx_wait(b):
            pltpu.make_async_copy(
                si_hbm.at[pl.ds(0, CHUNK)], sidx_v.at[b], sem_si.at[b]
            ).wait()
            pltpu.make_async_copy(
                ri_hbm.at[pl.ds(0, CHUNK)], ridx_v.at[b], sem_ri.at[b]
            ).wait()

        def gather_start(b):
            pltpu.make_async_copy(
                s_hbm.at[sidx_v.at[b]], srow_v.at[b], sem_sg.at[b]
            ).start()
            pltpu.make_async_copy(
                r_hbm.at[ridx_v.at[b]], rrow_v.at[b], sem_rg.at[b]
            ).start()

        def gather_wait(b):
            pltpu.make_async_copy(
                s_hbm.at[sidx_v.at[b]], srow_v.at[b], sem_sg.at[b]
            ).wait()
            pltpu.make_async_copy(
                r_hbm.at[ridx_v.at[b]], rrow_v.at[b], sem_rg.at[b]
            ).wait()

        def out_start(k, b):
            base = base_w + k * CHUNK
            pltpu.make_async_copy(
                out_v.at[b], out_hbm.at[pl.ds(base * DD, CHUNK * DD)], sem_out.at[b]
            ).start()

        def out_wait(b):
            pltpu.make_async_copy(
                out_v.at[b], out_hbm.at[pl.ds(0, CHUNK * DD)], sem_out.at[b]
            ).wait()

        def compute(b):
            def emit(c, s_vec, r_rep):
                c64 = c * DD
                for k in range(4):
                    s_pat = _lane_shuffle(s_vec, s_pat_idx[k])
                    out_v[b, pl.ds(c64 + k * 16, 16)] = s_pat * r_rep

            def edge_body(c, carry):
                s_vec, r_rep = carry
                nxt = (srow_v[b, c + 1], rrow_v[b, c + 1])
                emit(c, s_vec, r_rep)
                return nxt

            first = (srow_v[b, 0], rrow_v[b, 0])
            last = lax.fori_loop(0, CHUNK - 1, edge_body, first, unroll=8)
            emit(CHUNK - 1, *last)

        idx_start(0, 0)
        idx_start(1, 1)
        idx_wait(0)
        gather_start(0)

        def pair_body(k2, carry):
            for b in (0, 1):
                k = 2 * k2 + b

                @pl.when(k < n_chunks)
                def _():
                    @pl.when(k + 1 < n_chunks)
                    def _():
                        idx_wait(1 - b)
                        gather_start(1 - b)

                    gather_wait(b)

                    @pl.when(k + 2 < n_chunks)
                    def _():
                        idx_start(k + 2, b)

                    @pl.when(k >= 2)
                    def _():
                        out_wait(b)

                    compute(b)
                    out_start(k, b)

            return carry

        lax.fori_loop(0, (n_chunks + 1) // 2, pair_body, 0, unroll=False)
        out_wait(0)
        out_wait(1)

    return pl.kernel(
        body,
        out_type=jax.ShapeDtypeStruct((n_edges * DD,), jnp.float32),
        mesh=mesh,
        compiler_params=pltpu.CompilerParams(
            needs_layout_passes=False, use_tc_tiling_on_sc=False
        ),
        scratch_types=[
            pltpu.VMEM((2, CHUNK), jnp.int32),
            pltpu.VMEM((2, CHUNK), jnp.int32),
            pltpu.VMEM((2, CHUNK, DP), jnp.float32),
            pltpu.VMEM((2, CHUNK, DP), jnp.float32),
            pltpu.VMEM((2, CHUNK * DD), jnp.float32),
            pltpu.SemaphoreType.DMA((2,)),
            pltpu.SemaphoreType.DMA((2,)),
            pltpu.SemaphoreType.DMA((2,)),
            pltpu.SemaphoreType.DMA((2,)),
            pltpu.SemaphoreType.DMA((2,)),
        ],
    )


def kernel(edge_index, node_type_s, node_type_r=None):
    if node_type_r is None:
        node_type_r = node_type_s
    n_edges = edge_index.shape[1]
    s_p = jnp.pad(node_type_s, ((0, 0), (0, DP - D)))
    r_p = jnp.concatenate([node_type_r, node_type_r], axis=1)
    f = _make(n_edges)
    out_flat = f(edge_index[0], edge_index[1], s_p, r_p)
    return out_flat.reshape(n_edges, DD)
